# baseline (speedup 1.0000x reference)
import functools

import numpy as np

import jax
import jax.numpy as jnp
from jax import lax
from jax.experimental import pallas as pl
from jax.experimental.pallas import tpu as pltpu
from jax.sharding import Mesh, PartitionSpec as P

try:
    from jax import shard_map as _shard_map
except ImportError:
    from jax.experimental.shard_map import shard_map as _shard_map

_FILL = 4


def _make_kernel(tk):
    dims = (((1,), (1,)), ((), ()))

    def _matmul_kernel(x_ref, w_ref, o_ref, w_bf):
        i = pl.program_id(1)

        @pl.when(i < _FILL)
        def _():
            w_bf[:, pl.ds(i * tk, tk)] = w_ref[...].astype(jnp.bfloat16)
            x_slab = x_ref[:, pl.ds(i * tk, tk)].astype(jnp.bfloat16)
            partial = lax.dot_general(
                x_slab,
                w_bf[:, pl.ds(i * tk, tk)],
                dimension_numbers=dims,
                preferred_element_type=jnp.float32,
            )

            @pl.when(i == 0)
            def _():
                o_ref[...] = partial

            @pl.when(i != 0)
            def _():
                o_ref[...] += partial

        @pl.when(i >= _FILL)
        def _():
            x = x_ref[...].astype(jnp.bfloat16)
            o_ref[...] = lax.dot_general(
                x,
                w_bf[...],
                dimension_numbers=dims,
                preferred_element_type=jnp.float32,
            )

    return _matmul_kernel


@functools.partial(jax.jit, static_argnames=("tm",))
def _my_linear(x2, weight, tm):
    M, K = x2.shape
    N = weight.shape[0]
    tn = N // 2
    tk = K // _FILL

    grid = (2, _FILL - 1 + M // tm)

    cost = pl.CostEstimate(
        flops=2 * M * N * K,
        bytes_accessed=4 * M * K + 4 * N * K + 4 * M * N,
        transcendentals=0,
    )

    return pl.pallas_call(
        _make_kernel(tk),
        out_shape=jax.ShapeDtypeStruct((M, N), jnp.float32),
        grid=grid,
        in_specs=[
            pl.BlockSpec(
                (tm, K), lambda j, i: (jnp.maximum(i - (_FILL - 1), 0), 0)
            ),
            pl.BlockSpec(
                (tn, tk), lambda j, i: (j, jnp.minimum(i, _FILL - 1))
            ),
        ],
        out_specs=pl.BlockSpec(
            (tm, tn), lambda j, i: (jnp.maximum(i - (_FILL - 1), 0), j)
        ),
        scratch_shapes=[pltpu.VMEM((tn, K), jnp.bfloat16)],
        compiler_params=pltpu.CompilerParams(
            dimension_semantics=("arbitrary", "arbitrary"),
            vmem_limit_bytes=64 * 1024 * 1024,
        ),
        cost_estimate=cost,
    )(x2, weight)


def kernel(x, weight):
    orig_shape = x.shape
    K = orig_shape[-1]
    x2 = x.reshape(-1, K)
    N = weight.shape[0]
    M = x2.shape[0]
    devs = jax.devices()
    if len(devs) >= 2 and M % (2 * 512) == 0:
        mesh = Mesh(np.array(devs[:2]), ("m",))
        out = _shard_map(
            functools.partial(_my_linear, tm=512),
            mesh=mesh,
            in_specs=(P("m", None), P(None, None)),
            out_specs=P("m", None),
            check_vma=False,
        )(x2, weight)
    else:
        out = _my_linear(x2, weight, tm=512)
    return out.reshape(orig_shape[:-1] + (N,))


# speedup vs baseline: 2.4450x; 2.4450x over previous
import functools

import jax
import jax.numpy as jnp
from jax import lax
from jax.experimental import pallas as pl
from jax.experimental.pallas import tpu as pltpu

_FILL = 4


def _make_kernel(tk):
    dims = (((1,), (1,)), ((), ()))

    def _matmul_kernel(x_ref, w_ref, o_ref, w_bf):
        i = pl.program_id(1)

        @pl.when(i < _FILL)
        def _():
            w_bf[:, pl.ds(i * tk, tk)] = w_ref[...].astype(jnp.bfloat16)
            x_slab = x_ref[:, pl.ds(i * tk, tk)].astype(jnp.bfloat16)
            partial = lax.dot_general(
                x_slab,
                w_bf[:, pl.ds(i * tk, tk)],
                dimension_numbers=dims,
                preferred_element_type=jnp.float32,
            )

            @pl.when(i == 0)
            def _():
                o_ref[...] = partial

            @pl.when(i != 0)
            def _():
                o_ref[...] += partial

        @pl.when(i >= _FILL)
        def _():
            x = x_ref[...].astype(jnp.bfloat16)
            o_ref[...] = lax.dot_general(
                x,
                w_bf[...],
                dimension_numbers=dims,
                preferred_element_type=jnp.float32,
            )

    return _matmul_kernel


@functools.partial(jax.jit, static_argnames=("tm",))
def _my_linear(x2, weight, tm):
    M, K = x2.shape
    N = weight.shape[0]
    tn = N // 2
    tk = K // _FILL

    grid = (2, _FILL - 1 + M // tm)

    cost = pl.CostEstimate(
        flops=2 * M * N * K,
        bytes_accessed=4 * M * K + 4 * N * K + 4 * M * N,
        transcendentals=0,
    )

    return pl.pallas_call(
        _make_kernel(tk),
        out_shape=jax.ShapeDtypeStruct((M, N), jnp.float32),
        grid=grid,
        in_specs=[
            pl.BlockSpec(
                (tm, K), lambda j, i: (jnp.maximum(i - (_FILL - 1), 0), 0)
            ),
            pl.BlockSpec(
                (tn, tk), lambda j, i: (j, jnp.minimum(i, _FILL - 1))
            ),
        ],
        out_specs=pl.BlockSpec(
            (tm, tn), lambda j, i: (jnp.maximum(i - (_FILL - 1), 0), j)
        ),
        scratch_shapes=[pltpu.VMEM((tn, K), jnp.bfloat16)],
        compiler_params=pltpu.CompilerParams(
            dimension_semantics=("parallel", "arbitrary"),
            vmem_limit_bytes=64 * 1024 * 1024,
        ),
        cost_estimate=cost,
    )(x2, weight)


def kernel(x, weight):
    orig_shape = x.shape
    K = orig_shape[-1]
    x2 = x.reshape(-1, K)
    N = weight.shape[0]
    out = _my_linear(x2, weight, tm=512)
    return out.reshape(orig_shape[:-1] + (N,))
